# baseline (speedup 1.0000x reference)
import functools

import jax
import jax.numpy as jnp
from jax import lax
from jax.experimental import pallas as pl
from jax.experimental.pallas import tpu as pltpu
from jax.experimental.pallas import tpu_sc as plsc

N = 10000
D = 128
E = 320000
NPAD = 10112
RPT = NPAD // 16
CHS = (128, 128, 128, 128, 120)
NC, NS = 2, 16
NW = NC * NS
K = 128
NB = 80
EP = NW * NB * K
RB = 2528
HB = 40
MLP_HID = 200
MLP_PAD = 256
BN_EPS = 1e-5

_mesh = plsc.VectorSubcoreMesh(
    core_axis_name="c", subcore_axis_name="s", num_cores=NC, num_subcores=NS)


def _sc_body_segsum(p_hbm, srcb, dstb, z128,
                    part,
                    acc, srch, dsth, rows0, sem0):
    c = lax.axis_index("c")
    s = lax.axis_index("s")
    wid = s * NC + c
    r0 = s * RPT
    pltpu.sync_copy(z128, rows0)
    off = 0
    for ch in CHS:
        pltpu.sync_copy(rows0.at[pl.ds(0, ch)], acc.at[pl.ds(r0 + off, ch)])
        off += ch
    plsc.subcore_barrier()

    @pl.loop(0, NB)
    def _batch(j):
        pltpu.sync_copy(srcb.at[wid, j], srch)
        pltpu.sync_copy(dstb.at[wid, j], dsth)
        pltpu.async_copy(p_hbm.at[srch], rows0, sem0).wait()
        pltpu.sync_copy(rows0, acc.at[dsth], add=True)

    plsc.subcore_barrier()
    off = 0
    for ch in CHS:
        pltpu.sync_copy(acc.at[pl.ds(r0 + off, ch)], rows0.at[pl.ds(0, ch)])
        pltpu.sync_copy(rows0.at[pl.ds(0, ch)],
                        part.at[c, pl.ds(r0 + off, ch)])
        off += ch


def _sc_body_deg(dstb, z128, onesb,
                 degp,
                 dacc, dsth, w_v, sem):
    c = lax.axis_index("c")
    s = lax.axis_index("s")
    wid = s * NC + c
    r0 = s * RPT
    pltpu.sync_copy(z128, w_v)
    off = 0
    for ch in CHS:
        pltpu.sync_copy(w_v.at[pl.ds(0, ch)], dacc.at[pl.ds(r0 + off, ch)])
        off += ch
    pltpu.sync_copy(onesb, w_v)
    plsc.subcore_barrier()

    for h in range(2):
        pltpu.sync_copy(dstb.at[wid, pl.ds(h * HB, HB)], dsth)

        @pl.loop(0, HB)
        def _batch(j):
            pltpu.sync_copy(w_v, dacc.at[dsth.at[j]], add=True)

    plsc.subcore_barrier()
    off = 0
    for ch in CHS:
        pltpu.sync_copy(dacc.at[pl.ds(r0 + off, ch)], w_v.at[pl.ds(0, ch)])
        pltpu.sync_copy(w_v.at[pl.ds(0, ch)],
                        degp.at[c, pl.ds(r0 + off, ch)])
        off += ch


_segsum = functools.partial(
    pl.kernel,
    out_type=jax.ShapeDtypeStruct((NC, NPAD, D), jnp.float32),
    mesh=_mesh,
    scratch_types=[
        pltpu.VMEM_SHARED((NPAD, D), jnp.float32),
        pltpu.VMEM((K,), jnp.int32),
        pltpu.VMEM((K,), jnp.int32),
        pltpu.VMEM((K, D), jnp.float32),
        pltpu.SemaphoreType.DMA,
    ],
)(_sc_body_segsum)

_degree = functools.partial(
    pl.kernel,
    out_type=jax.ShapeDtypeStruct((NC, NPAD, D), jnp.float32),
    mesh=_mesh,
    scratch_types=[
        pltpu.VMEM_SHARED((NPAD, D), jnp.float32),
        pltpu.VMEM((HB, K), jnp.int32),
        pltpu.VMEM((K, D), jnp.float32),
        pltpu.SemaphoreType.DMA,
    ],
)(_sc_body_deg)


def _mm_body(x_ref, w_ref, o_ref):
    o_ref[...] = jnp.dot(x_ref[...], w_ref[...],
                         preferred_element_type=jnp.float32)


def _matmul(x, w):
    return pl.pallas_call(
        _mm_body,
        grid=(NPAD // RB,),
        in_specs=[pl.BlockSpec((RB, D), lambda i: (i, 0)),
                  pl.BlockSpec((D, D), lambda i: (0, 0))],
        out_specs=pl.BlockSpec((RB, D), lambda i: (i, 0)),
        out_shape=jax.ShapeDtypeStruct((NPAD, D), jnp.float32),
    )(x, w)


def _combine_body(pa_ref, pb_ref, p_ref, da_ref, db_ref, b_ref, w_ref, o_ref):
    deg = da_ref[:, 0:1] + db_ref[:, 0:1] + 2.0
    s = pa_ref[...] + pb_ref[...] + 2.0 * p_ref[...]
    h = jnp.maximum(s / deg + b_ref[...], 0.0)
    o_ref[...] = jnp.dot(h, w_ref[...], preferred_element_type=jnp.float32)


def _combine_mm(pa, pb, p, da, db, b, w):
    return pl.pallas_call(
        _combine_body,
        grid=(NPAD // RB,),
        in_specs=[pl.BlockSpec((RB, D), lambda i: (i, 0)),
                  pl.BlockSpec((RB, D), lambda i: (i, 0)),
                  pl.BlockSpec((RB, D), lambda i: (i, 0)),
                  pl.BlockSpec((RB, D), lambda i: (i, 0)),
                  pl.BlockSpec((RB, D), lambda i: (i, 0)),
                  pl.BlockSpec((1, D), lambda i: (0, 0)),
                  pl.BlockSpec((D, D), lambda i: (0, 0))],
        out_specs=pl.BlockSpec((RB, D), lambda i: (i, 0)),
        out_shape=jax.ShapeDtypeStruct((NPAD, D), jnp.float32),
    )(pa, pb, p, da, db, b, w)


def _head_body(pa_ref, pb_ref, p_ref, da_ref, db_ref, b_ref,
               wm1_ref, bm1_ref, g_ref, beta_ref, wm2_ref, bm2_ref, o_ref):
    deg = da_ref[:, 0:1] + db_ref[:, 0:1] + 2.0
    s = pa_ref[...] + pb_ref[...] + 2.0 * p_ref[...]
    h2 = jnp.maximum(s / deg + b_ref[...], 0.0)
    y = jnp.maximum(jnp.dot(h2, wm1_ref[...],
                            preferred_element_type=jnp.float32)
                    + bm1_ref[...], 0.0)
    y = y * (g_ref[...] * (1.0 / jnp.sqrt(1.0 + BN_EPS))) + beta_ref[...]
    o_ref[...] = jnp.dot(y, wm2_ref[...],
                         preferred_element_type=jnp.float32) + bm2_ref[...]


def _head(pa, pb, p, da, db, b, wm1, bm1, g, beta, wm2, bm2):
    return pl.pallas_call(
        _head_body,
        grid=(NPAD // RB,),
        in_specs=[pl.BlockSpec((RB, D), lambda i: (i, 0)),
                  pl.BlockSpec((RB, D), lambda i: (i, 0)),
                  pl.BlockSpec((RB, D), lambda i: (i, 0)),
                  pl.BlockSpec((RB, D), lambda i: (i, 0)),
                  pl.BlockSpec((RB, D), lambda i: (i, 0)),
                  pl.BlockSpec((1, D), lambda i: (0, 0)),
                  pl.BlockSpec((D, MLP_PAD), lambda i: (0, 0)),
                  pl.BlockSpec((1, MLP_PAD), lambda i: (0, 0)),
                  pl.BlockSpec((1, MLP_PAD), lambda i: (0, 0)),
                  pl.BlockSpec((1, MLP_PAD), lambda i: (0, 0)),
                  pl.BlockSpec((MLP_PAD, 2), lambda i: (0, 0)),
                  pl.BlockSpec((1, 2), lambda i: (0, 0))],
        out_specs=pl.BlockSpec((RB, 2), lambda i: (i, 0)),
        out_shape=jax.ShapeDtypeStruct((NPAD, 2), jnp.float32),
    )(pa, pb, p, da, db, b, wm1, bm1, g, beta, wm2, bm2)


def kernel(features, W1, b1, W2, b2, Wm1, bm1, gamma, beta, Wm2, bm2,
           edge_index):
    feats = jnp.zeros((NPAD, D), jnp.float32).at[:N].set(features)
    src = edge_index[0]
    dst = edge_index[1]
    pad = EP - E
    srcb = jnp.concatenate(
        [src, jnp.zeros((pad,), jnp.int32)]).reshape(NW, NB, K)
    spread = N + (jnp.arange(pad, dtype=jnp.int32) % (NPAD - N - 1))
    dstb = jnp.concatenate([dst, spread]).reshape(NW, NB, K)
    z128 = jnp.zeros((128, D), jnp.float32)
    onesb = jnp.ones((K, D), jnp.float32)

    b1r = b1.reshape(1, D)
    b2r = b2.reshape(1, D)
    wm1p = jnp.zeros((D, MLP_PAD), jnp.float32).at[:, :MLP_HID].set(Wm1)
    bm1p = jnp.zeros((1, MLP_PAD), jnp.float32).at[:, :MLP_HID].set(bm1)
    gp = jnp.zeros((1, MLP_PAD), jnp.float32).at[:, :MLP_HID].set(gamma)
    betap = jnp.zeros((1, MLP_PAD), jnp.float32).at[:, :MLP_HID].set(beta)
    wm2p = jnp.zeros((MLP_PAD, 2), jnp.float32).at[:MLP_HID].set(Wm2)
    bm2r = bm2.reshape(1, 2)

    p1 = _matmul(feats, W1)
    degp = _degree(dstb, z128, onesb)
    part1 = _segsum(p1, srcb, dstb, z128)
    p2 = _combine_mm(part1[0], part1[1], p1, degp[0], degp[1], b1r, W2)
    part2 = _segsum(p2, srcb, dstb, z128)
    pred = _head(part2[0], part2[1], p2, degp[0], degp[1], b2r,
                 wm1p, bm1p, gp, betap, wm2p, bm2r)
    return pred[:N]


# speedup vs baseline: 1.3259x; 1.3259x over previous
import functools

import jax
import jax.numpy as jnp
from jax import lax
from jax.experimental import pallas as pl
from jax.experimental.pallas import tpu as pltpu
from jax.experimental.pallas import tpu_sc as plsc

N = 10000
D = 128
E = 320000
NPAD = 10240
RPT = NPAD // 16
CHS = (128, 128, 128, 128, 128)
NC, NS = 2, 16
NW = NC * NS
K = 128
NB = 79
EP = NW * NB * K
RB = 2560
MLP_HID = 200
MLP_PAD = 256
BN_EPS = 1e-5

_mesh = plsc.VectorSubcoreMesh(
    core_axis_name="c", subcore_axis_name="s", num_cores=NC, num_subcores=NS)


def _sc_body_segsum(p_hbm, srcb, dstb, z128,
                    part,
                    acc, srch, dsth, rows0, sem0):
    c = lax.axis_index("c")
    s = lax.axis_index("s")
    wid = s * NC + c
    r0 = s * RPT
    pltpu.sync_copy(z128, rows0)
    off = 0
    for ch in CHS:
        pltpu.sync_copy(rows0.at[pl.ds(0, ch)], acc.at[pl.ds(r0 + off, ch)])
        off += ch
    plsc.subcore_barrier()

    @pl.loop(0, NB)
    def _batch(j):
        pltpu.sync_copy(srcb.at[wid, j], srch)
        pltpu.sync_copy(dstb.at[wid, j], dsth)
        pltpu.async_copy(p_hbm.at[srch], rows0, sem0).wait()
        pltpu.sync_copy(rows0, acc.at[dsth], add=True)

    plsc.subcore_barrier()
    off = 0
    for ch in CHS:
        pltpu.sync_copy(acc.at[pl.ds(r0 + off, ch)], rows0.at[pl.ds(0, ch)])
        pltpu.sync_copy(rows0.at[pl.ds(0, ch)],
                        part.at[c, pl.ds(r0 + off, ch)])
        off += ch


def _sc_body_deg(dstb, z128, onesb,
                 degp,
                 dacc, dsth, w_v, sem):
    c = lax.axis_index("c")
    s = lax.axis_index("s")
    wid = s * NC + c
    r0 = s * RPT
    pltpu.sync_copy(z128, w_v)
    off = 0
    for ch in CHS:
        pltpu.sync_copy(w_v.at[pl.ds(0, ch)], dacc.at[pl.ds(r0 + off, ch)])
        off += ch
    pltpu.sync_copy(onesb, w_v)
    plsc.subcore_barrier()

    @pl.loop(0, NB)
    def _batch(j):
        pltpu.sync_copy(dstb.at[wid, j], dsth)
        pltpu.sync_copy(w_v, dacc.at[dsth], add=True)

    plsc.subcore_barrier()
    off = 0
    for ch in CHS:
        pltpu.sync_copy(dacc.at[pl.ds(r0 + off, ch)], w_v.at[pl.ds(0, ch)])
        pltpu.sync_copy(w_v.at[pl.ds(0, ch)],
                        degp.at[c, pl.ds(r0 + off, ch)])
        off += ch


_segsum = functools.partial(
    pl.kernel,
    out_type=jax.ShapeDtypeStruct((NC, NPAD, D), jnp.float32),
    mesh=_mesh,
    scratch_types=[
        pltpu.VMEM_SHARED((NPAD, D), jnp.float32),
        pltpu.VMEM((K,), jnp.int32),
        pltpu.VMEM((K,), jnp.int32),
        pltpu.VMEM((K, D), jnp.float32),
        pltpu.SemaphoreType.DMA,
    ],
)(_sc_body_segsum)

_degree = functools.partial(
    pl.kernel,
    out_type=jax.ShapeDtypeStruct((NC, NPAD, D), jnp.float32),
    mesh=_mesh,
    scratch_types=[
        pltpu.VMEM_SHARED((NPAD, D), jnp.float32),
        pltpu.VMEM((K,), jnp.int32),
        pltpu.VMEM((K, D), jnp.float32),
        pltpu.SemaphoreType.DMA,
    ],
)(_sc_body_deg)


def _mm_body(x_ref, w_ref, o_ref):
    o_ref[...] = jnp.dot(x_ref[...], w_ref[...],
                         preferred_element_type=jnp.float32)


def _matmul(x, w):
    return pl.pallas_call(
        _mm_body,
        grid=(NPAD // RB,),
        in_specs=[pl.BlockSpec((RB, D), lambda i: (i, 0)),
                  pl.BlockSpec((D, D), lambda i: (0, 0))],
        out_specs=pl.BlockSpec((RB, D), lambda i: (i, 0)),
        out_shape=jax.ShapeDtypeStruct((NPAD, D), jnp.float32),
    )(x, w)


def _combine_body(pa_ref, pb_ref, p_ref, da_ref, db_ref, b_ref, w_ref, o_ref):
    deg = da_ref[:, 0:1] + db_ref[:, 0:1] + 2.0
    s = pa_ref[...] + pb_ref[...] + 2.0 * p_ref[...]
    h = jnp.maximum(s / deg + b_ref[...], 0.0)
    o_ref[...] = jnp.dot(h, w_ref[...], preferred_element_type=jnp.float32)


def _combine_mm(pa, pb, p, da, db, b, w):
    return pl.pallas_call(
        _combine_body,
        grid=(NPAD // RB,),
        in_specs=[pl.BlockSpec((RB, D), lambda i: (i, 0)),
                  pl.BlockSpec((RB, D), lambda i: (i, 0)),
                  pl.BlockSpec((RB, D), lambda i: (i, 0)),
                  pl.BlockSpec((RB, D), lambda i: (i, 0)),
                  pl.BlockSpec((RB, D), lambda i: (i, 0)),
                  pl.BlockSpec((1, D), lambda i: (0, 0)),
                  pl.BlockSpec((D, D), lambda i: (0, 0))],
        out_specs=pl.BlockSpec((RB, D), lambda i: (i, 0)),
        out_shape=jax.ShapeDtypeStruct((NPAD, D), jnp.float32),
    )(pa, pb, p, da, db, b, w)


def _head_body(pa_ref, pb_ref, p_ref, da_ref, db_ref, b_ref,
               wm1_ref, bm1_ref, g_ref, beta_ref, wm2_ref, bm2_ref, o_ref):
    deg = da_ref[:, 0:1] + db_ref[:, 0:1] + 2.0
    s = pa_ref[...] + pb_ref[...] + 2.0 * p_ref[...]
    h2 = jnp.maximum(s / deg + b_ref[...], 0.0)
    y = jnp.maximum(jnp.dot(h2, wm1_ref[...],
                            preferred_element_type=jnp.float32)
                    + bm1_ref[...], 0.0)
    y = y * (g_ref[...] * (1.0 / jnp.sqrt(1.0 + BN_EPS))) + beta_ref[...]
    o_ref[...] = jnp.dot(y, wm2_ref[...],
                         preferred_element_type=jnp.float32) + bm2_ref[...]


def _head(pa, pb, p, da, db, b, wm1, bm1, g, beta, wm2, bm2):
    return pl.pallas_call(
        _head_body,
        grid=(NPAD // RB,),
        in_specs=[pl.BlockSpec((RB, D), lambda i: (i, 0)),
                  pl.BlockSpec((RB, D), lambda i: (i, 0)),
                  pl.BlockSpec((RB, D), lambda i: (i, 0)),
                  pl.BlockSpec((RB, D), lambda i: (i, 0)),
                  pl.BlockSpec((RB, D), lambda i: (i, 0)),
                  pl.BlockSpec((1, D), lambda i: (0, 0)),
                  pl.BlockSpec((D, MLP_PAD), lambda i: (0, 0)),
                  pl.BlockSpec((1, MLP_PAD), lambda i: (0, 0)),
                  pl.BlockSpec((1, MLP_PAD), lambda i: (0, 0)),
                  pl.BlockSpec((1, MLP_PAD), lambda i: (0, 0)),
                  pl.BlockSpec((MLP_PAD, 2), lambda i: (0, 0)),
                  pl.BlockSpec((1, 2), lambda i: (0, 0))],
        out_specs=pl.BlockSpec((RB, 2), lambda i: (i, 0)),
        out_shape=jax.ShapeDtypeStruct((NPAD, 2), jnp.float32),
    )(pa, pb, p, da, db, b, wm1, bm1, g, beta, wm2, bm2)


def kernel(features, W1, b1, W2, b2, Wm1, bm1, gamma, beta, Wm2, bm2,
           edge_index):
    feats = jnp.zeros((NPAD, D), jnp.float32).at[:N].set(features)
    src = edge_index[0]
    dst = edge_index[1]
    pad = EP - E
    srcb = jnp.concatenate(
        [src, jnp.zeros((pad,), jnp.int32)]).reshape(NW, NB, K)
    dstb = jnp.concatenate(
        [dst, jnp.full((pad,), N, jnp.int32)]).reshape(NW, NB, K)
    z128 = jnp.zeros((128, D), jnp.float32)
    onesb = jnp.ones((K, D), jnp.float32)

    b1r = b1.reshape(1, D)
    b2r = b2.reshape(1, D)
    wm1p = jnp.zeros((D, MLP_PAD), jnp.float32).at[:, :MLP_HID].set(Wm1)
    bm1p = jnp.zeros((1, MLP_PAD), jnp.float32).at[:, :MLP_HID].set(bm1)
    gp = jnp.zeros((1, MLP_PAD), jnp.float32).at[:, :MLP_HID].set(gamma)
    betap = jnp.zeros((1, MLP_PAD), jnp.float32).at[:, :MLP_HID].set(beta)
    wm2p = jnp.zeros((MLP_PAD, 2), jnp.float32).at[:MLP_HID].set(Wm2)
    bm2r = bm2.reshape(1, 2)

    p1 = _matmul(feats, W1)
    degp = _degree(dstb, z128, onesb)
    part1 = _segsum(p1, srcb, dstb, z128)
    p2 = _combine_mm(part1[0], part1[1], p1, degp[0], degp[1], b1r, W2)
    part2 = _segsum(p2, srcb, dstb, z128)
    pred = _head(part2[0], part2[1], p2, degp[0], degp[1], b2r,
                 wm1p, bm1p, gp, betap, wm2p, bm2r)
    return pred[:N]


# speedup vs baseline: 1.5079x; 1.1373x over previous
import functools

import jax
import jax.numpy as jnp
from jax import lax
from jax.experimental import pallas as pl
from jax.experimental.pallas import tpu as pltpu
from jax.experimental.pallas import tpu_sc as plsc

N = 10000
D = 128
E = 320000
NPAD = 10240
RPT = NPAD // 16
CHS = (128, 128, 128, 128, 128)
NC, NS = 2, 16
NW = NC * NS
K = 128
NB = 79
EP = NW * NB * K
RB = 2560
MLP_HID = 200
MLP_PAD = 256
BN_EPS = 1e-5

_mesh = plsc.VectorSubcoreMesh(
    core_axis_name="c", subcore_axis_name="s", num_cores=NC, num_subcores=NS)


def _sc_body_segsum(p_hbm, srcb, dstb, z128,
                    part,
                    acc, src0, src1, dst0, dst1, rows0, isem0, isem1, sem0):
    c = lax.axis_index("c")
    s = lax.axis_index("s")
    wid = s * NC + c
    r0 = s * RPT
    pltpu.sync_copy(z128, rows0)
    off = 0
    for ch in CHS:
        pltpu.sync_copy(rows0.at[pl.ds(0, ch)], acc.at[pl.ds(r0 + off, ch)])
        off += ch
    plsc.subcore_barrier()

    bufs = ((src0, dst0, isem0), (src1, dst1, isem1))

    def fire_idx(j, b):
        src_b, dst_b, isem_b = bufs[b]
        pltpu.async_copy(srcb.at[wid, j], src_b, isem_b)
        pltpu.async_copy(dstb.at[wid, j], dst_b, isem_b)

    fire_idx(0, 0)
    fire_idx(1, 1)

    @pl.loop(0, NB)
    def _batch(j):
        for b in range(2):

            @pl.when(lax.rem(j, 2) == b)
            def _():
                src_b, dst_b, isem_b = bufs[b]
                pltpu.make_async_copy(srcb.at[wid, j], src_b, isem_b).wait()
                pltpu.make_async_copy(dstb.at[wid, j], dst_b, isem_b).wait()
                pltpu.async_copy(p_hbm.at[src_b], rows0, sem0).wait()
                pltpu.sync_copy(rows0, acc.at[dst_b], add=True)

                @pl.when(j + 2 < NB)
                def _():
                    fire_idx(j + 2, b)

    plsc.subcore_barrier()
    off = 0
    for ch in CHS:
        pltpu.sync_copy(acc.at[pl.ds(r0 + off, ch)], rows0.at[pl.ds(0, ch)])
        pltpu.sync_copy(rows0.at[pl.ds(0, ch)],
                        part.at[c, pl.ds(r0 + off, ch)])
        off += ch


def _sc_body_deg(dstb, z128, onesb,
                 degp,
                 dacc, dsth, w_v, sem):
    c = lax.axis_index("c")
    s = lax.axis_index("s")
    wid = s * NC + c
    r0 = s * RPT
    pltpu.sync_copy(z128, w_v)
    off = 0
    for ch in CHS:
        pltpu.sync_copy(w_v.at[pl.ds(0, ch)], dacc.at[pl.ds(r0 + off, ch)])
        off += ch
    pltpu.sync_copy(onesb, w_v)
    plsc.subcore_barrier()

    @pl.loop(0, NB)
    def _batch(j):
        pltpu.sync_copy(dstb.at[wid, j], dsth)
        pltpu.sync_copy(w_v, dacc.at[dsth], add=True)

    plsc.subcore_barrier()
    off = 0
    for ch in CHS:
        pltpu.sync_copy(dacc.at[pl.ds(r0 + off, ch)], w_v.at[pl.ds(0, ch)])
        pltpu.sync_copy(w_v.at[pl.ds(0, ch)],
                        degp.at[c, pl.ds(r0 + off, ch)])
        off += ch


_segsum = functools.partial(
    pl.kernel,
    out_type=jax.ShapeDtypeStruct((NC, NPAD, D), jnp.float32),
    mesh=_mesh,
    scratch_types=[
        pltpu.VMEM_SHARED((NPAD, D), jnp.float32),
        pltpu.VMEM((K,), jnp.int32),
        pltpu.VMEM((K,), jnp.int32),
        pltpu.VMEM((K,), jnp.int32),
        pltpu.VMEM((K,), jnp.int32),
        pltpu.VMEM((K, D), jnp.float32),
        pltpu.SemaphoreType.DMA,
        pltpu.SemaphoreType.DMA,
        pltpu.SemaphoreType.DMA,
    ],
)(_sc_body_segsum)

_degree = functools.partial(
    pl.kernel,
    out_type=jax.ShapeDtypeStruct((NC, NPAD, D), jnp.float32),
    mesh=_mesh,
    scratch_types=[
        pltpu.VMEM_SHARED((NPAD, D), jnp.float32),
        pltpu.VMEM((K,), jnp.int32),
        pltpu.VMEM((K, D), jnp.float32),
        pltpu.SemaphoreType.DMA,
    ],
)(_sc_body_deg)


def _mm_body(x_ref, w_ref, o_ref):
    o_ref[...] = jnp.dot(x_ref[...], w_ref[...],
                         preferred_element_type=jnp.float32)


def _matmul(x, w):
    return pl.pallas_call(
        _mm_body,
        grid=(NPAD // RB,),
        in_specs=[pl.BlockSpec((RB, D), lambda i: (i, 0)),
                  pl.BlockSpec((D, D), lambda i: (0, 0))],
        out_specs=pl.BlockSpec((RB, D), lambda i: (i, 0)),
        out_shape=jax.ShapeDtypeStruct((NPAD, D), jnp.float32),
    )(x, w)


def _combine_body(pa_ref, pb_ref, p_ref, da_ref, db_ref, b_ref, w_ref, o_ref):
    deg = da_ref[:, 0:1] + db_ref[:, 0:1] + 2.0
    s = pa_ref[...] + pb_ref[...] + 2.0 * p_ref[...]
    h = jnp.maximum(s / deg + b_ref[...], 0.0)
    o_ref[...] = jnp.dot(h, w_ref[...], preferred_element_type=jnp.float32)


def _combine_mm(pa, pb, p, da, db, b, w):
    return pl.pallas_call(
        _combine_body,
        grid=(NPAD // RB,),
        in_specs=[pl.BlockSpec((RB, D), lambda i: (i, 0)),
                  pl.BlockSpec((RB, D), lambda i: (i, 0)),
                  pl.BlockSpec((RB, D), lambda i: (i, 0)),
                  pl.BlockSpec((RB, D), lambda i: (i, 0)),
                  pl.BlockSpec((RB, D), lambda i: (i, 0)),
                  pl.BlockSpec((1, D), lambda i: (0, 0)),
                  pl.BlockSpec((D, D), lambda i: (0, 0))],
        out_specs=pl.BlockSpec((RB, D), lambda i: (i, 0)),
        out_shape=jax.ShapeDtypeStruct((NPAD, D), jnp.float32),
    )(pa, pb, p, da, db, b, w)


def _head_body(pa_ref, pb_ref, p_ref, da_ref, db_ref, b_ref,
               wm1_ref, bm1_ref, g_ref, beta_ref, wm2_ref, bm2_ref, o_ref):
    deg = da_ref[:, 0:1] + db_ref[:, 0:1] + 2.0
    s = pa_ref[...] + pb_ref[...] + 2.0 * p_ref[...]
    h2 = jnp.maximum(s / deg + b_ref[...], 0.0)
    y = jnp.maximum(jnp.dot(h2, wm1_ref[...],
                            preferred_element_type=jnp.float32)
                    + bm1_ref[...], 0.0)
    y = y * (g_ref[...] * (1.0 / jnp.sqrt(1.0 + BN_EPS))) + beta_ref[...]
    o_ref[...] = jnp.dot(y, wm2_ref[...],
                         preferred_element_type=jnp.float32) + bm2_ref[...]


def _head(pa, pb, p, da, db, b, wm1, bm1, g, beta, wm2, bm2):
    return pl.pallas_call(
        _head_body,
        grid=(NPAD // RB,),
        in_specs=[pl.BlockSpec((RB, D), lambda i: (i, 0)),
                  pl.BlockSpec((RB, D), lambda i: (i, 0)),
                  pl.BlockSpec((RB, D), lambda i: (i, 0)),
                  pl.BlockSpec((RB, D), lambda i: (i, 0)),
                  pl.BlockSpec((RB, D), lambda i: (i, 0)),
                  pl.BlockSpec((1, D), lambda i: (0, 0)),
                  pl.BlockSpec((D, MLP_PAD), lambda i: (0, 0)),
                  pl.BlockSpec((1, MLP_PAD), lambda i: (0, 0)),
                  pl.BlockSpec((1, MLP_PAD), lambda i: (0, 0)),
                  pl.BlockSpec((1, MLP_PAD), lambda i: (0, 0)),
                  pl.BlockSpec((MLP_PAD, 2), lambda i: (0, 0)),
                  pl.BlockSpec((1, 2), lambda i: (0, 0))],
        out_specs=pl.BlockSpec((RB, 2), lambda i: (i, 0)),
        out_shape=jax.ShapeDtypeStruct((NPAD, 2), jnp.float32),
    )(pa, pb, p, da, db, b, wm1, bm1, g, beta, wm2, bm2)


def kernel(features, W1, b1, W2, b2, Wm1, bm1, gamma, beta, Wm2, bm2,
           edge_index):
    feats = jnp.zeros((NPAD, D), jnp.float32).at[:N].set(features)
    src = edge_index[0]
    dst = edge_index[1]
    pad = EP - E
    srcb = jnp.concatenate(
        [src, jnp.zeros((pad,), jnp.int32)]).reshape(NW, NB, K)
    dstb = jnp.concatenate(
        [dst, jnp.full((pad,), N, jnp.int32)]).reshape(NW, NB, K)
    z128 = jnp.zeros((128, D), jnp.float32)
    onesb = jnp.ones((K, D), jnp.float32)

    b1r = b1.reshape(1, D)
    b2r = b2.reshape(1, D)
    wm1p = jnp.zeros((D, MLP_PAD), jnp.float32).at[:, :MLP_HID].set(Wm1)
    bm1p = jnp.zeros((1, MLP_PAD), jnp.float32).at[:, :MLP_HID].set(bm1)
    gp = jnp.zeros((1, MLP_PAD), jnp.float32).at[:, :MLP_HID].set(gamma)
    betap = jnp.zeros((1, MLP_PAD), jnp.float32).at[:, :MLP_HID].set(beta)
    wm2p = jnp.zeros((MLP_PAD, 2), jnp.float32).at[:MLP_HID].set(Wm2)
    bm2r = bm2.reshape(1, 2)

    p1 = _matmul(feats, W1)
    degp = _degree(dstb, z128, onesb)
    part1 = _segsum(p1, srcb, dstb, z128)
    p2 = _combine_mm(part1[0], part1[1], p1, degp[0], degp[1], b1r, W2)
    part2 = _segsum(p2, srcb, dstb, z128)
    pred = _head(part2[0], part2[1], p2, degp[0], degp[1], b2r,
                 wm1p, bm1p, gp, betap, wm2p, bm2r)
    return pred[:N]


# speedup vs baseline: 1.5613x; 1.0354x over previous
import functools

import jax
import jax.numpy as jnp
from jax import lax
from jax.experimental import pallas as pl
from jax.experimental.pallas import tpu as pltpu
from jax.experimental.pallas import tpu_sc as plsc

N = 10000
D = 128
E = 320000
NPAD = 10240
RPT = NPAD // 16
CHS = (128, 128, 128, 128, 128)
NC, NS = 2, 16
NW = NC * NS
K = 128
NB = 79
EP = NW * NB * K
RB = 2560
MLP_HID = 200
MLP_PAD = 256
BN_EPS = 1e-5

_mesh = plsc.VectorSubcoreMesh(
    core_axis_name="c", subcore_axis_name="s", num_cores=NC, num_subcores=NS)


def _sc_body_segsum(p_hbm, srcb, dstb, z128,
                    part,
                    acc, src0, src1, dst0, dst1, rows0, isem0, isem1, sem0):
    c = lax.axis_index("c")
    s = lax.axis_index("s")
    wid = s * NC + c
    r0 = s * RPT
    pltpu.sync_copy(z128, rows0)
    off = 0
    for ch in CHS:
        pltpu.sync_copy(rows0.at[pl.ds(0, ch)], acc.at[pl.ds(r0 + off, ch)])
        off += ch
    plsc.subcore_barrier()

    bufs = ((src0, dst0, isem0), (src1, dst1, isem1))

    def fire_idx(j, b):
        src_b, dst_b, isem_b = bufs[b]
        pltpu.async_copy(srcb.at[wid, j], src_b, isem_b)
        pltpu.async_copy(dstb.at[wid, j], dst_b, isem_b)

    fire_idx(0, 0)
    fire_idx(1, 1)

    @pl.loop(0, NB)
    def _batch(j):
        for b in range(2):

            @pl.when(lax.rem(j, 2) == b)
            def _():
                src_b, dst_b, isem_b = bufs[b]
                pltpu.make_async_copy(srcb.at[wid, j], src_b, isem_b).wait()
                pltpu.make_async_copy(dstb.at[wid, j], dst_b, isem_b).wait()
                pltpu.async_copy(p_hbm.at[src_b], rows0, sem0).wait()
                pltpu.sync_copy(rows0, acc.at[dst_b], add=True)

                @pl.when(j + 2 < NB)
                def _():
                    fire_idx(j + 2, b)

    plsc.subcore_barrier()
    off = 0
    for ch in CHS:
        pltpu.sync_copy(acc.at[pl.ds(r0 + off, ch)], rows0.at[pl.ds(0, ch)])
        pltpu.sync_copy(rows0.at[pl.ds(0, ch)],
                        part.at[c, pl.ds(r0 + off, ch)])
        off += ch


def _sc_body_deg(dstb, z128, onesb,
                 degp,
                 dacc, dst0, dst1, w_v, isem0, isem1):
    c = lax.axis_index("c")
    s = lax.axis_index("s")
    wid = s * NC + c
    r0 = s * RPT
    pltpu.sync_copy(z128, w_v)
    off = 0
    for ch in CHS:
        pltpu.sync_copy(w_v.at[pl.ds(0, ch)], dacc.at[pl.ds(r0 + off, ch)])
        off += ch
    pltpu.sync_copy(onesb, w_v)
    plsc.subcore_barrier()

    dbufs = ((dst0, isem0), (dst1, isem1))
    pltpu.async_copy(dstb.at[wid, 0], dst0, isem0)
    pltpu.async_copy(dstb.at[wid, 1], dst1, isem1)

    @pl.loop(0, NB)
    def _batch(j):
        for b in range(2):

            @pl.when(lax.rem(j, 2) == b)
            def _():
                dst_b, isem_b = dbufs[b]
                pltpu.make_async_copy(dstb.at[wid, j], dst_b, isem_b).wait()
                pltpu.sync_copy(w_v, dacc.at[dst_b], add=True)

                @pl.when(j + 2 < NB)
                def _():
                    pltpu.async_copy(dstb.at[wid, j + 2], dst_b, isem_b)

    plsc.subcore_barrier()
    off = 0
    for ch in CHS:
        pltpu.sync_copy(dacc.at[pl.ds(r0 + off, ch)], w_v.at[pl.ds(0, ch)])
        pltpu.sync_copy(w_v.at[pl.ds(0, ch)],
                        degp.at[c, pl.ds(r0 + off, ch)])
        off += ch


_segsum = functools.partial(
    pl.kernel,
    out_type=jax.ShapeDtypeStruct((NC, NPAD, D), jnp.float32),
    mesh=_mesh,
    scratch_types=[
        pltpu.VMEM_SHARED((NPAD, D), jnp.float32),
        pltpu.VMEM((K,), jnp.int32),
        pltpu.VMEM((K,), jnp.int32),
        pltpu.VMEM((K,), jnp.int32),
        pltpu.VMEM((K,), jnp.int32),
        pltpu.VMEM((K, D), jnp.float32),
        pltpu.SemaphoreType.DMA,
        pltpu.SemaphoreType.DMA,
        pltpu.SemaphoreType.DMA,
    ],
)(_sc_body_segsum)

_degree = functools.partial(
    pl.kernel,
    out_type=jax.ShapeDtypeStruct((NC, NPAD, D), jnp.float32),
    mesh=_mesh,
    scratch_types=[
        pltpu.VMEM_SHARED((NPAD, D), jnp.float32),
        pltpu.VMEM((K,), jnp.int32),
        pltpu.VMEM((K,), jnp.int32),
        pltpu.VMEM((K, D), jnp.float32),
        pltpu.SemaphoreType.DMA,
        pltpu.SemaphoreType.DMA,
    ],
)(_sc_body_deg)


def _mm_body(x_ref, w_ref, o_ref):
    o_ref[...] = jnp.dot(x_ref[...], w_ref[...],
                         preferred_element_type=jnp.float32)


def _matmul(x, w):
    return pl.pallas_call(
        _mm_body,
        grid=(NPAD // RB,),
        in_specs=[pl.BlockSpec((RB, D), lambda i: (i, 0)),
                  pl.BlockSpec((D, D), lambda i: (0, 0))],
        out_specs=pl.BlockSpec((RB, D), lambda i: (i, 0)),
        out_shape=jax.ShapeDtypeStruct((NPAD, D), jnp.float32),
    )(x, w)


def _combine_body(pa_ref, pb_ref, p_ref, da_ref, db_ref, b_ref, w_ref, o_ref):
    deg = da_ref[:, 0:1] + db_ref[:, 0:1] + 2.0
    s = pa_ref[...] + pb_ref[...] + 2.0 * p_ref[...]
    h = jnp.maximum(s / deg + b_ref[...], 0.0)
    o_ref[...] = jnp.dot(h, w_ref[...], preferred_element_type=jnp.float32)


def _combine_mm(pa, pb, p, da, db, b, w):
    return pl.pallas_call(
        _combine_body,
        grid=(NPAD // RB,),
        in_specs=[pl.BlockSpec((RB, D), lambda i: (i, 0)),
                  pl.BlockSpec((RB, D), lambda i: (i, 0)),
                  pl.BlockSpec((RB, D), lambda i: (i, 0)),
                  pl.BlockSpec((RB, D), lambda i: (i, 0)),
                  pl.BlockSpec((RB, D), lambda i: (i, 0)),
                  pl.BlockSpec((1, D), lambda i: (0, 0)),
                  pl.BlockSpec((D, D), lambda i: (0, 0))],
        out_specs=pl.BlockSpec((RB, D), lambda i: (i, 0)),
        out_shape=jax.ShapeDtypeStruct((NPAD, D), jnp.float32),
    )(pa, pb, p, da, db, b, w)


def _head_body(pa_ref, pb_ref, p_ref, da_ref, db_ref, b_ref,
               wm1_ref, bm1_ref, g_ref, beta_ref, wm2_ref, bm2_ref, o_ref):
    deg = da_ref[:, 0:1] + db_ref[:, 0:1] + 2.0
    s = pa_ref[...] + pb_ref[...] + 2.0 * p_ref[...]
    h2 = jnp.maximum(s / deg + b_ref[...], 0.0)
    y = jnp.maximum(jnp.dot(h2, wm1_ref[...],
                            preferred_element_type=jnp.float32)
                    + bm1_ref[...], 0.0)
    y = y * (g_ref[...] * (1.0 / jnp.sqrt(1.0 + BN_EPS))) + beta_ref[...]
    o_ref[...] = jnp.dot(y, wm2_ref[...],
                         preferred_element_type=jnp.float32) + bm2_ref[...]


def _head(pa, pb, p, da, db, b, wm1, bm1, g, beta, wm2, bm2):
    return pl.pallas_call(
        _head_body,
        grid=(NPAD // RB,),
        in_specs=[pl.BlockSpec((RB, D), lambda i: (i, 0)),
                  pl.BlockSpec((RB, D), lambda i: (i, 0)),
                  pl.BlockSpec((RB, D), lambda i: (i, 0)),
                  pl.BlockSpec((RB, D), lambda i: (i, 0)),
                  pl.BlockSpec((RB, D), lambda i: (i, 0)),
                  pl.BlockSpec((1, D), lambda i: (0, 0)),
                  pl.BlockSpec((D, MLP_PAD), lambda i: (0, 0)),
                  pl.BlockSpec((1, MLP_PAD), lambda i: (0, 0)),
                  pl.BlockSpec((1, MLP_PAD), lambda i: (0, 0)),
                  pl.BlockSpec((1, MLP_PAD), lambda i: (0, 0)),
                  pl.BlockSpec((MLP_PAD, 2), lambda i: (0, 0)),
                  pl.BlockSpec((1, 2), lambda i: (0, 0))],
        out_specs=pl.BlockSpec((RB, 2), lambda i: (i, 0)),
        out_shape=jax.ShapeDtypeStruct((NPAD, 2), jnp.float32),
    )(pa, pb, p, da, db, b, wm1, bm1, g, beta, wm2, bm2)


def kernel(features, W1, b1, W2, b2, Wm1, bm1, gamma, beta, Wm2, bm2,
           edge_index):
    feats = jnp.zeros((NPAD, D), jnp.float32).at[:N].set(features)
    src = edge_index[0]
    dst = edge_index[1]
    pad = EP - E
    srcb = jnp.concatenate(
        [src, jnp.zeros((pad,), jnp.int32)]).reshape(NW, NB, K)
    dstb = jnp.concatenate(
        [dst, jnp.full((pad,), N, jnp.int32)]).reshape(NW, NB, K)
    z128 = jnp.zeros((128, D), jnp.float32)
    onesb = jnp.ones((K, D), jnp.float32)

    b1r = b1.reshape(1, D)
    b2r = b2.reshape(1, D)
    wm1p = jnp.zeros((D, MLP_PAD), jnp.float32).at[:, :MLP_HID].set(Wm1)
    bm1p = jnp.zeros((1, MLP_PAD), jnp.float32).at[:, :MLP_HID].set(bm1)
    gp = jnp.zeros((1, MLP_PAD), jnp.float32).at[:, :MLP_HID].set(gamma)
    betap = jnp.zeros((1, MLP_PAD), jnp.float32).at[:, :MLP_HID].set(beta)
    wm2p = jnp.zeros((MLP_PAD, 2), jnp.float32).at[:MLP_HID].set(Wm2)
    bm2r = bm2.reshape(1, 2)

    p1 = _matmul(feats, W1)
    degp = _degree(dstb, z128, onesb)
    part1 = _segsum(p1, srcb, dstb, z128)
    p2 = _combine_mm(part1[0], part1[1], p1, degp[0], degp[1], b1r, W2)
    part2 = _segsum(p2, srcb, dstb, z128)
    pred = _head(part2[0], part2[1], p2, degp[0], degp[1], b2r,
                 wm1p, bm1p, gp, betap, wm2p, bm2r)
    return pred[:N]
